# baseline (speedup 1.0000x reference)
import functools

import jax
import jax.numpy as jnp
from jax import lax
from jax.experimental import pallas as pl
from jax.experimental.pallas import tpu as pltpu
from jax.experimental.pallas import tpu_sc as plsc

_USER_NUM = 1000000
_ITEM_NUM = 100000
_FACTOR = 32
_BATCH = 16384
_LAMDA = 0.001

_NC = 2
_NS = 16
_NW = _NC * _NS
_BPW = _BATCH // _NW
_CHUNK = 64
_NCHUNK = _BPW // _CHUNK
_NBLK = _CHUNK // 16


def _sc_body(uidx, iidx, puidx, piidx, nuidx, niidx, ridx, nridx,
             utab, itab, urtab, irtab,
             xhat_out, sq_out,
             u_v, i_v, pu_v, pi_v, nu_v, ni_v, r_v, nr_v,
             sup0, sup1, sup2, sup3, sup4, sup5,
             sup6, sup7, sup8, sup9, sup10, sup11,
             sub0, sub1, sub2, sub3, sub4, sub5,
             sub6, sub7, sub8, sub9, sub10, sub11,
             d0, d1, d2, d3, d4, d5, d6, d7, d8, d9, d10, d11,
             xh_v, s0, s1, s2, s3, s4, s5, s6, s7, s8, s9, s10, s11,
             sem):
    wid = lax.axis_index("s") * _NC + lax.axis_index("c")
    base = wid * _BPW

    pltpu.sync_copy(uidx.at[pl.ds(base, _BPW)], u_v)
    pltpu.sync_copy(iidx.at[pl.ds(base, _BPW)], i_v)
    pltpu.sync_copy(puidx.at[pl.ds(base, _BPW)], pu_v)
    pltpu.sync_copy(piidx.at[pl.ds(base, _BPW)], pi_v)
    pltpu.sync_copy(nuidx.at[pl.ds(base, _BPW)], nu_v)
    pltpu.sync_copy(niidx.at[pl.ds(base, _BPW)], ni_v)
    pltpu.sync_copy(ridx.at[pl.ds(base, _BPW)], r_v)
    pltpu.sync_copy(nridx.at[pl.ds(base, _BPW)], nr_v)

    sups = (sup0, sup1, sup2, sup3, sup4, sup5,
            sup6, sup7, sup8, sup9, sup10, sup11)
    subs = (sub0, sub1, sub2, sub3, sub4, sub5,
            sub6, sub7, sub8, sub9, sub10, sub11)
    dbufs = (d0, d1, d2, d3, d4, d5, d6, d7, d8, d9, d10, d11)
    sbufs = (s0, s1, s2, s3, s4, s5, s6, s7, s8, s9, s10, s11)
    tabs = (utab, urtab, itab, irtab,
            utab, urtab, itab, irtab,
            utab, urtab, itab, irtab)

    def ibody(i, _):
        sl = pl.ds(i * 16, 16)
        r = r_v[sl]
        nr = nr_v[sl]
        raw = (u_v[sl], u_v[sl] + r * _USER_NUM,
               i_v[sl], i_v[sl] + r * _ITEM_NUM,
               pu_v[sl], pu_v[sl] + r * _USER_NUM,
               pi_v[sl], pi_v[sl] + r * _ITEM_NUM,
               nu_v[sl], nu_v[sl] + nr * _USER_NUM,
               ni_v[sl], ni_v[sl] + nr * _ITEM_NUM)
        for k in range(12):
            sups[k][sl] = lax.shift_right_logical(raw[k], 2)
            subs[k][sl] = lax.shift_left(raw[k] & 3, 5)
        return 0

    lax.fori_loop(0, _BPW // 16, ibody, 0)

    iota16 = lax.iota(jnp.int32, 16)

    for chunk in range(_NCHUNK):
        coff = chunk * _CHUNK
        copies = [
            pltpu.async_copy(tabs[k].at[sups[k].at[pl.ds(coff, _CHUNK)]],
                             dbufs[k], sem)
            for k in range(12)
        ]
        for cp in copies:
            cp.wait()

        def bbody(b, _):
            rowi = b * 16 + iota16
            off = coff + b * 16
            sl = pl.ds(off, 16)
            subv = [subs[k][sl] for k in range(12)]

            def cbody(c, accs):
                v = [plsc.load_gather(dbufs[k], [rowi, subv[k] + c])
                     for k in range(12)]
                cg = (v[0] + v[1]) + (v[2] + v[3])
                cd = ((v[4] + v[5]) + (v[6] + v[7])) - \
                     ((v[8] + v[9]) + (v[10] + v[11]))
                new = [accs[0] + cg * cd]
                for k in range(12):
                    new.append(accs[k + 1] + v[k] * v[k])
                return tuple(new)

            zero = jnp.zeros((16,), jnp.float32)
            accs = lax.fori_loop(0, _FACTOR, cbody, (zero,) * 13)
            xh_v[sl] = accs[0]
            for k in range(12):
                sbufs[k][sl] = accs[k + 1]
            return 0

        lax.fori_loop(0, _NBLK, bbody, 0)

    pltpu.sync_copy(xh_v, xhat_out.at[pl.ds(base, _BPW)])
    for k in range(12):
        pltpu.sync_copy(sbufs[k], sq_out.at[pl.ds(k * _BATCH + base, _BPW)])


_sc_kernel = functools.partial(
    pl.kernel,
    mesh=plsc.VectorSubcoreMesh(core_axis_name="c", subcore_axis_name="s"),
    out_type=[
        jax.ShapeDtypeStruct((_BATCH,), jnp.float32),
        jax.ShapeDtypeStruct((12 * _BATCH,), jnp.float32),
    ],
    scratch_types=(
        [pltpu.VMEM((_BPW,), jnp.int32) for _ in range(8 + 24)]
        + [pltpu.VMEM((_CHUNK, 128), jnp.float32) for _ in range(12)]
        + [pltpu.VMEM((_BPW,), jnp.float32) for _ in range(13)]
        + [pltpu.SemaphoreType.DMA]
    ),
    compiler_params=pltpu.CompilerParams(needs_layout_passes=False),
)(_sc_body)


def _tc_body(xhat_ref, sq_ref, loss_ref, reg_ref):
    x = xhat_ref[...]
    ls = jnp.maximum(-x, 0.0) + jnp.log1p(jnp.exp(-jnp.abs(x)))
    loss_ref[0, 0] = jnp.sum(ls)
    reg_ref[0, 0] = _LAMDA * jnp.sum(jnp.sqrt(sq_ref[...]))


_tc_kernel = pl.pallas_call(
    _tc_body,
    out_shape=[
        jax.ShapeDtypeStruct((1, 1), jnp.float32),
        jax.ShapeDtypeStruct((1, 1), jnp.float32),
    ],
    out_specs=[
        pl.BlockSpec(memory_space=pltpu.SMEM),
        pl.BlockSpec(memory_space=pltpu.SMEM),
    ],
)


def kernel(user_idx, item_idx, pos_user_idx, pos_item_idx, neg_user_idx,
           neg_item_idx, rel_idx, neg_rel_idx, user_table, item_table,
           urel_table, irel_table):
    xhat, sq = _sc_kernel(
        user_idx.astype(jnp.int32), item_idx.astype(jnp.int32),
        pos_user_idx.astype(jnp.int32), pos_item_idx.astype(jnp.int32),
        neg_user_idx.astype(jnp.int32), neg_item_idx.astype(jnp.int32),
        rel_idx.astype(jnp.int32), neg_rel_idx.astype(jnp.int32),
        user_table.reshape(_USER_NUM // 4, 128),
        item_table.reshape(_ITEM_NUM // 4, 128),
        urel_table.reshape(3 * _USER_NUM // 4, 128),
        irel_table.reshape(3 * _ITEM_NUM // 4, 128))
    loss, reg = _tc_kernel(xhat.reshape(128, 128),
                           sq.reshape(192, 1024))
    return (loss[0, 0], reg[0, 0])


# speedup vs baseline: 1.0268x; 1.0268x over previous
import functools

import jax
import jax.numpy as jnp
from jax import lax
from jax.experimental import pallas as pl
from jax.experimental.pallas import tpu as pltpu
from jax.experimental.pallas import tpu_sc as plsc

_USER_NUM = 1000000
_ITEM_NUM = 100000
_FACTOR = 32
_BATCH = 16384
_LAMDA = 0.001

_NC = 2
_NS = 16
_NW = _NC * _NS
_BPW = _BATCH // _NW
_CHUNK = 128
_NCHUNK = _BPW // _CHUNK
_NBLK = _CHUNK // 16


def _sc_body(uidx, iidx, puidx, piidx, nuidx, niidx, ridx, nridx,
             utab, itab, urtab, irtab,
             xhat_out, sq_out,
             u_v, i_v, pu_v, pi_v, nu_v, ni_v, r_v, nr_v,
             ur_v, ir_v, pur_v, pir_v, nur_v, nir_v,
             d0, d1, d2, d3, d4, d5, d6, d7, d8, d9, d10, d11,
             xh_v, s0, s1, s2, s3, s4, s5, s6, s7, s8, s9, s10, s11,
             sem):
    wid = lax.axis_index("s") * _NC + lax.axis_index("c")
    base = wid * _BPW

    pltpu.sync_copy(uidx.at[pl.ds(base, _BPW)], u_v)
    pltpu.sync_copy(iidx.at[pl.ds(base, _BPW)], i_v)
    pltpu.sync_copy(puidx.at[pl.ds(base, _BPW)], pu_v)
    pltpu.sync_copy(piidx.at[pl.ds(base, _BPW)], pi_v)
    pltpu.sync_copy(nuidx.at[pl.ds(base, _BPW)], nu_v)
    pltpu.sync_copy(niidx.at[pl.ds(base, _BPW)], ni_v)
    pltpu.sync_copy(ridx.at[pl.ds(base, _BPW)], r_v)
    pltpu.sync_copy(nridx.at[pl.ds(base, _BPW)], nr_v)

    def ibody(i, _):
        sl = pl.ds(i * 16, 16)
        r = r_v[sl]
        nr = nr_v[sl]
        ur_v[sl] = u_v[sl] + r * _USER_NUM
        ir_v[sl] = i_v[sl] + r * _ITEM_NUM
        pur_v[sl] = pu_v[sl] + r * _USER_NUM
        pir_v[sl] = pi_v[sl] + r * _ITEM_NUM
        nur_v[sl] = nu_v[sl] + nr * _USER_NUM
        nir_v[sl] = ni_v[sl] + nr * _ITEM_NUM
        return 0

    lax.fori_loop(0, _BPW // 16, ibody, 0)

    dbufs = (d0, d1, d2, d3, d4, d5, d6, d7, d8, d9, d10, d11)
    sbufs = (s0, s1, s2, s3, s4, s5, s6, s7, s8, s9, s10, s11)
    tabs = (utab, urtab, itab, irtab,
            utab, urtab, itab, irtab,
            utab, urtab, itab, irtab)
    ibufs = (u_v, ur_v, i_v, ir_v,
             pu_v, pur_v, pi_v, pir_v,
             nu_v, nur_v, ni_v, nir_v)

    iota16 = lax.iota(jnp.int32, 16)

    for chunk in range(_NCHUNK):
        coff = chunk * _CHUNK
        copies = [
            pltpu.async_copy(tabs[k].at[ibufs[k].at[pl.ds(coff, _CHUNK)]],
                             dbufs[k], sem)
            for k in range(12)
        ]
        for cp in copies:
            cp.wait()

        def bbody(b, _):
            rowi = b * 16 + iota16

            def cbody(c, accs):
                coli = jnp.full((16,), c, jnp.int32)
                v = [plsc.load_gather(dbufs[k], [rowi, coli])
                     for k in range(12)]
                cg = (v[0] + v[1]) + (v[2] + v[3])
                cd = ((v[4] + v[5]) + (v[6] + v[7])) - \
                     ((v[8] + v[9]) + (v[10] + v[11]))
                new = [accs[0] + cg * cd]
                for k in range(12):
                    new.append(accs[k + 1] + v[k] * v[k])
                return tuple(new)

            zero = jnp.zeros((16,), jnp.float32)
            accs = lax.fori_loop(0, _FACTOR, cbody, (zero,) * 13)
            off = coff + b * 16
            xh_v[pl.ds(off, 16)] = accs[0]
            for k in range(12):
                sbufs[k][pl.ds(off, 16)] = accs[k + 1]
            return 0

        lax.fori_loop(0, _NBLK, bbody, 0)

    pltpu.sync_copy(xh_v, xhat_out.at[pl.ds(base, _BPW)])
    for k in range(12):
        pltpu.sync_copy(sbufs[k], sq_out.at[pl.ds(k * _BATCH + base, _BPW)])


_sc_kernel = functools.partial(
    pl.kernel,
    mesh=plsc.VectorSubcoreMesh(core_axis_name="c", subcore_axis_name="s"),
    out_type=[
        jax.ShapeDtypeStruct((_BATCH,), jnp.float32),
        jax.ShapeDtypeStruct((12 * _BATCH,), jnp.float32),
    ],
    scratch_types=(
        [pltpu.VMEM((_BPW,), jnp.int32) for _ in range(14)]
        + [pltpu.VMEM((_CHUNK, _FACTOR), jnp.float32) for _ in range(12)]
        + [pltpu.VMEM((_BPW,), jnp.float32) for _ in range(13)]
        + [pltpu.SemaphoreType.DMA]
    ),
    compiler_params=pltpu.CompilerParams(needs_layout_passes=False,
                                         use_tc_tiling_on_sc=False),
)(_sc_body)


def _tc_body(xhat_ref, sq_ref, loss_ref, reg_ref):
    x = xhat_ref[...]
    ls = jnp.maximum(-x, 0.0) + jnp.log1p(jnp.exp(-jnp.abs(x)))
    loss_ref[0, 0] = jnp.sum(ls)
    reg_ref[0, 0] = _LAMDA * jnp.sum(jnp.sqrt(sq_ref[...]))


_tc_kernel = pl.pallas_call(
    _tc_body,
    out_shape=[
        jax.ShapeDtypeStruct((1, 1), jnp.float32),
        jax.ShapeDtypeStruct((1, 1), jnp.float32),
    ],
    out_specs=[
        pl.BlockSpec(memory_space=pltpu.SMEM),
        pl.BlockSpec(memory_space=pltpu.SMEM),
    ],
)


def kernel(user_idx, item_idx, pos_user_idx, pos_item_idx, neg_user_idx,
           neg_item_idx, rel_idx, neg_rel_idx, user_table, item_table,
           urel_table, irel_table):
    one = (user_idx[0] * 0 + 1).astype(jnp.float32)
    xhat, sq = _sc_kernel(
        user_idx.astype(jnp.int32), item_idx.astype(jnp.int32),
        pos_user_idx.astype(jnp.int32), pos_item_idx.astype(jnp.int32),
        neg_user_idx.astype(jnp.int32), neg_item_idx.astype(jnp.int32),
        rel_idx.astype(jnp.int32), neg_rel_idx.astype(jnp.int32),
        user_table * one, item_table * one,
        urel_table * one, irel_table * one)
    loss, reg = _tc_kernel(xhat.reshape(128, 128),
                           sq.reshape(192, 1024))
    return (loss[0, 0], reg[0, 0])
